# baseline (speedup 1.0000x reference)
import functools

import jax
import jax.numpy as jnp
import numpy as np
from jax import lax
from jax.experimental import pallas as pl
from jax.experimental.pallas import tpu as pltpu
from jax.experimental.pallas import tpu_sc as plsc

N = 10000
SENT = N
NP = 10112
E = 160000
EP = 163840
NC = 2
NS = 16
NW = NC * NS
CHUNK = 128
EPW = EP // NW
NCH = EPW // CHUNK
RPT = NP // NS
HROWS = 2560
HCH = HROWS // CHUNK
KG = 10

_R_NP = np.kron(np.eye(16, dtype=np.float32), np.ones((1, 16), np.float32))
_T32_NP = np.concatenate(
    [np.kron(np.ones((16, 1), np.float32), np.eye(16, dtype=np.float32)),
     np.zeros((256, 16), np.float32)], axis=1)


def _sc_front(net_feat, node_feat, src_pinned, src_near, src_pins, zeros32,
              ones32):
    mesh = plsc.VectorSubcoreMesh(core_axis_name="c", subcore_axis_name="s")

    @functools.partial(
        pl.kernel,
        out_type=[
            jax.ShapeDtypeStruct((EP, 16), jnp.float32),
            jax.ShapeDtypeStruct((EP, 16), jnp.float32),
            jax.ShapeDtypeStruct((NC, NP, 8), jnp.float32),
        ],
        mesh=mesh,
        scratch_types=[
            pltpu.VMEM((NCH, CHUNK), jnp.int32),
            pltpu.VMEM((EPW, 16), jnp.float32),
            pltpu.VMEM((CHUNK, 8), jnp.float32),
            pltpu.VMEM((KG * CHUNK, 8), jnp.float32),
            pltpu.VMEM_SHARED((NP, 8), jnp.float32),
            pltpu.SemaphoreType.DMA,
        ],
        compiler_params=pltpu.CompilerParams(use_tc_tiling_on_sc=False),
    )
    def k(net_hbm, node_hbm, src_pinned_hbm, src_near_hbm, src_pins_hbm,
          zeros_hbm, ones_hbm, gpinned_hbm, gnear_hbm, deg_hbm,
          idx_v, rows_v, ones_v, cnt_v, acc, sem):
        cid = lax.axis_index("c")
        sid = lax.axis_index("s")
        wid = sid * NC + cid
        pltpu.sync_copy(zeros_hbm.at[pl.ds(sid * RPT, RPT)],
                        acc.at[pl.ds(sid * RPT, RPT)])
        pltpu.sync_copy(ones_hbm, ones_v)
        plsc.subcore_barrier()

        def gather(src_hbm, table_hbm, out_hbm):
            pltpu.sync_copy(src_hbm.at[wid], idx_v)

            def gbody(g, carry):
                def fire(j, c2):
                    pltpu.async_copy(
                        table_hbm.at[idx_v.at[g * KG + j]],
                        rows_v.at[pl.ds((g * KG + j) * CHUNK, CHUNK)], sem)
                    return c2

                lax.fori_loop(0, KG, fire, 0)
                pltpu.make_async_copy(
                    out_hbm.at[pl.ds(0, KG * CHUNK)],
                    rows_v.at[pl.ds(g * KG * CHUNK, KG * CHUNK)], sem).wait()
                return carry

            lax.fori_loop(0, NCH // KG, gbody, 0)
            pltpu.sync_copy(rows_v, out_hbm.at[pl.ds(wid * EPW, EPW)])

        gather(src_pinned_hbm, net_hbm, gpinned_hbm)
        gather(src_near_hbm, node_hbm, gnear_hbm)

        pltpu.sync_copy(src_pins_hbm.at[wid], idx_v)

        def cgroup(g, carry):
            def fire(j, c2):
                pltpu.async_copy(ones_v, acc.at[idx_v.at[g * KG + j]], sem,
                                 add=True)
                return c2

            lax.fori_loop(0, KG, fire, 0)
            pltpu.make_async_copy(
                zeros_hbm.at[pl.ds(0, KG * CHUNK)], cnt_v, sem).wait()
            return carry

        lax.fori_loop(0, NCH // KG, cgroup, 0)
        plsc.subcore_barrier()
        pltpu.sync_copy(acc.at[pl.ds(sid * RPT, RPT)],
                        deg_hbm.at[cid, pl.ds(sid * RPT, RPT)])

    return k(net_feat, node_feat, src_pinned, src_near, src_pins, zeros32,
             ones32)


def _sc_agg(x32, msg_p, msg_n, src_pins, dst_pins, dst_pinned, dst_near,
            zeros32):
    mesh = plsc.VectorSubcoreMesh(core_axis_name="c", subcore_axis_name="s")

    @functools.partial(
        pl.kernel,
        out_type=[jax.ShapeDtypeStruct((NC, 3, NP, 32), jnp.float32)],
        mesh=mesh,
        scratch_types=[
            pltpu.VMEM((NCH, CHUNK), jnp.int32),
            pltpu.VMEM((NCH, CHUNK), jnp.int32),
            pltpu.VMEM((HROWS, 32), jnp.float32),
            pltpu.VMEM_SHARED((NP, 32), jnp.float32),
            pltpu.SemaphoreType.DMA,
        ],
        compiler_params=pltpu.CompilerParams(use_tc_tiling_on_sc=False),
    )
    def k(x32_hbm, msg_p_hbm, msg_n_hbm, src_pins_hbm, dst_pins_hbm,
          dst_pinned_hbm, dst_near_hbm, zeros_hbm, out_hbm,
          idx_s, idx_d, rows_v, acc, sem):
        cid = lax.axis_index("c")
        sid = lax.axis_index("s")
        wid = sid * NC + cid

        def zero_acc():
            pltpu.sync_copy(zeros_hbm.at[pl.ds(sid * RPT, RPT)],
                            acc.at[pl.ds(sid * RPT, RPT)])
            plsc.subcore_barrier()

        def flush_acc(r):
            plsc.subcore_barrier()
            pltpu.sync_copy(acc.at[pl.ds(sid * RPT, RPT)],
                            out_hbm.at[cid, r, pl.ds(sid * RPT, RPT)])

        def drain(n_rows):
            pltpu.make_async_copy(x32_hbm.at[pl.ds(0, n_rows)],
                                  rows_v.at[pl.ds(0, n_rows)], sem).wait()

        zero_acc()
        pltpu.sync_copy(src_pins_hbm.at[wid], idx_s)
        pltpu.sync_copy(dst_pins_hbm.at[wid], idx_d)

        def pphase(q, carry):
            def fire_g(j, c2):
                pltpu.async_copy(x32_hbm.at[idx_s.at[q * KG + j]],
                                 rows_v.at[pl.ds(j * CHUNK, CHUNK)], sem)
                return c2

            lax.fori_loop(0, KG, fire_g, 0)
            drain(KG * CHUNK)

            def fire_s(j, c2):
                pltpu.async_copy(rows_v.at[pl.ds(j * CHUNK, CHUNK)],
                                 acc.at[idx_d.at[q * KG + j]], sem, add=True)
                return c2

            lax.fori_loop(0, KG, fire_s, 0)
            drain(KG * CHUNK)
            return carry

        lax.fori_loop(0, NCH // KG, pphase, 0)
        flush_acc(0)

        def scat(msg_hbm):
            def hbody(h, carry):
                pltpu.sync_copy(
                    msg_hbm.at[pl.ds(wid * EPW + h * HROWS, HROWS)], rows_v)

                def fire_s(j, c2):
                    pltpu.async_copy(rows_v.at[pl.ds(j * CHUNK, CHUNK)],
                                     acc.at[idx_d.at[h * HCH + j]], sem,
                                     add=True)
                    return c2

                lax.fori_loop(0, HCH, fire_s, 0)
                drain(HROWS)
                return carry

            lax.fori_loop(0, EPW // HROWS, hbody, 0)

        zero_acc()
        pltpu.sync_copy(dst_pinned_hbm.at[wid], idx_d)
        scat(msg_p_hbm)
        flush_acc(1)

        zero_acc()
        pltpu.sync_copy(dst_near_hbm.at[wid], idx_d)
        scat(msg_n_hbm)
        flush_acc(2)

    (out,) = k(x32, msg_p, msg_n, src_pins, dst_pins, dst_pinned, dst_near,
               zeros32)
    return out


def _tc_scale(nf_pad, deg_parts):
    def body(nf_ref, d_ref, o_ref):
        deg8 = d_ref[0] + d_ref[1]
        deg = jnp.concatenate([deg8, deg8], axis=1)
        x16 = nf_ref[...] * lax.rsqrt(jnp.maximum(deg, 1.0))
        o_ref[...] = jnp.concatenate(
            [x16, jnp.ones((NP, 16), jnp.float32)], axis=1)

    return pl.pallas_call(
        body, out_shape=jax.ShapeDtypeStruct((NP, 32), jnp.float32),
    )(nf_pad, deg_parts)


def _tc_msg(g, ef, w_lin, b_lin, r_c, t_c, blk):

    nblk = E // blk

    def body(g_ref, ef_ref, w_ref, b_ref, r_ref, t_ref, o_ref):
        w_e = jnp.dot(ef_ref[...], w_ref[...],
                      preferred_element_type=jnp.float32) + b_ref[...]
        fx = jnp.dot(g_ref[...], r_ref[...],
                     preferred_element_type=jnp.float32)
        m = jnp.dot(w_e * fx, t_ref[...], preferred_element_type=jnp.float32)
        col = lax.broadcasted_iota(jnp.int32, (blk, 32), 1)
        o_ref[...] = m + (col >= 16).astype(jnp.float32)

    return pl.pallas_call(
        body,
        grid=(nblk,),
        in_specs=[
            pl.BlockSpec((blk, 16), lambda i: (i, 0)),
            pl.BlockSpec((blk, 16), lambda i: (i, 0)),
            pl.BlockSpec((16, 256), lambda i: (0, 0)),
            pl.BlockSpec((1, 256), lambda i: (0, 0)),
            pl.BlockSpec((16, 256), lambda i: (0, 0)),
            pl.BlockSpec((256, 32), lambda i: (0, 0)),
        ],
        out_specs=pl.BlockSpec((blk, 32), lambda i: (i, 0)),
        out_shape=jax.ShapeDtypeStruct((EP, 32), jnp.float32),
    )(g, ef, w_lin, b_lin, r_c, t_c)


def _tc_final(parts, w_gc, b_gc, b_pinned, b_near):
    def body(p_ref, w_ref, bg_ref, bp_ref, bn_ref, node_ref, net_ref):
        agg = p_ref[0, 0, :, :16] + p_ref[1, 0, :, :16]
        deg_in = p_ref[0, 0, :, 16:32] + p_ref[1, 0, :, 16:32]
        rst = agg * lax.rsqrt(jnp.maximum(deg_in, 1.0))
        net_ref[...] = jnp.dot(rst, w_ref[...],
                               preferred_element_type=jnp.float32) + bg_ref[...]
        s1 = p_ref[0, 1, :, :16] + p_ref[1, 1, :, :16]
        c1 = p_ref[0, 1, :, 16:32] + p_ref[1, 1, :, 16:32]
        o1 = s1 / jnp.maximum(c1, 1.0) + bp_ref[...]
        s2 = p_ref[0, 2, :, :16] + p_ref[1, 2, :, :16]
        c2 = p_ref[0, 2, :, 16:32] + p_ref[1, 2, :, 16:32]
        o2 = s2 / jnp.maximum(c2, 1.0) + bn_ref[...]
        node_ref[...] = jnp.maximum(o1, o2)

    return pl.pallas_call(
        body,
        out_shape=[jax.ShapeDtypeStruct((NP, 16), jnp.float32),
                   jax.ShapeDtypeStruct((NP, 16), jnp.float32)],
    )(parts, w_gc, b_gc, b_pinned, b_near)


def kernel(node_feat, net_feat, pin_feat, edge_feat, pins_edge_index,
           pinned_edge_index, near_edge_index, w_gc, b_gc, w_topo, b_topo,
           w_geom, b_geom, b_pinned, b_near):
    f32 = jnp.float32

    def prep_idx(a, fill):
        pad = jnp.full((EP - E,), fill, jnp.int32)
        return jnp.concatenate([a.astype(jnp.int32), pad]).reshape(
            NW, NCH, CHUNK)

    def pad_rows(a, n):
        return jnp.concatenate(
            [a, jnp.zeros((n - a.shape[0], a.shape[1]), a.dtype)])

    src_pins = prep_idx(pins_edge_index[0], SENT)
    dst_pins = prep_idx(pins_edge_index[1], SENT)
    src_pinned = prep_idx(pinned_edge_index[0], 0)
    dst_pinned = prep_idx(pinned_edge_index[1], SENT)
    src_near = prep_idx(near_edge_index[0], 0)
    dst_near = prep_idx(near_edge_index[1], SENT)

    zeros32 = jnp.zeros((NP, 32), f32)
    zeros8 = jnp.zeros((NP, 8), f32)
    ones8 = jnp.ones((CHUNK, 8), f32)
    nf_pad = pad_rows(node_feat, NP)
    r_c = jnp.asarray(_R_NP)
    t_c = jnp.asarray(_T32_NP)

    gpinned, gnear, deg_parts = _sc_front(
        net_feat, node_feat, src_pinned, src_near, src_pins, zeros8, ones8)
    x32 = _tc_scale(nf_pad, deg_parts)
    msg_p = _tc_msg(gpinned, pin_feat, w_topo, b_topo.reshape(1, 256), r_c,
                    t_c, 2000)
    msg_n = _tc_msg(gnear, edge_feat, w_geom, b_geom.reshape(1, 256), r_c,
                    t_c, 2000)
    parts = _sc_agg(x32, msg_p, msg_n, src_pins, dst_pins, dst_pinned,
                    dst_near, zeros32)
    node_out, net_out = _tc_final(parts, w_gc, b_gc.reshape(1, 16),
                                  b_pinned.reshape(1, 16),
                                  b_near.reshape(1, 16))
    return node_out[:N], net_out[:N]


# speedup vs baseline: 1.0737x; 1.0737x over previous
import functools

import jax
import jax.numpy as jnp
import numpy as np
from jax import lax
from jax.experimental import pallas as pl
from jax.experimental.pallas import tpu as pltpu
from jax.experimental.pallas import tpu_sc as plsc

N = 10000
SENT = N
NP = 10112
E = 160000
EP = 163840
NC = 2
NS = 16
NW = NC * NS
CHUNK = 128
EPW = EP // NW
NCH = EPW // CHUNK
RPT = NP // NS
HROWS = 2560
HCH = HROWS // CHUNK
KG = 10

W = 24
_R_NP = np.kron(np.eye(16, dtype=np.float32), np.ones((1, 16), np.float32))
_T24_NP = np.concatenate(
    [np.kron(np.ones((16, 1), np.float32), np.eye(16, dtype=np.float32)),
     np.zeros((256, 8), np.float32)], axis=1)


def _sc_front(net_feat, node_feat, src_pinned, src_near, src_pins, zeros32,
              ones32):
    mesh = plsc.VectorSubcoreMesh(core_axis_name="c", subcore_axis_name="s")

    @functools.partial(
        pl.kernel,
        out_type=[
            jax.ShapeDtypeStruct((EP, 16), jnp.float32),
            jax.ShapeDtypeStruct((EP, 16), jnp.float32),
            jax.ShapeDtypeStruct((NC, NP, 8), jnp.float32),
        ],
        mesh=mesh,
        scratch_types=[
            pltpu.VMEM((NCH, CHUNK), jnp.int32),
            pltpu.VMEM((EPW, 16), jnp.float32),
            pltpu.VMEM((CHUNK, 8), jnp.float32),
            pltpu.VMEM((KG * CHUNK, 8), jnp.float32),
            pltpu.VMEM_SHARED((NP, 8), jnp.float32),
            pltpu.SemaphoreType.DMA,
        ],
        compiler_params=pltpu.CompilerParams(use_tc_tiling_on_sc=False),
    )
    def k(net_hbm, node_hbm, src_pinned_hbm, src_near_hbm, src_pins_hbm,
          zeros_hbm, ones_hbm, gpinned_hbm, gnear_hbm, deg_hbm,
          idx_v, rows_v, ones_v, cnt_v, acc, sem):
        cid = lax.axis_index("c")
        sid = lax.axis_index("s")
        wid = sid * NC + cid
        pltpu.sync_copy(zeros_hbm.at[pl.ds(sid * RPT, RPT)],
                        acc.at[pl.ds(sid * RPT, RPT)])
        pltpu.sync_copy(ones_hbm, ones_v)
        plsc.subcore_barrier()

        def gather(src_hbm, table_hbm, out_hbm):
            pltpu.sync_copy(src_hbm.at[wid], idx_v)

            def gbody(g, carry):
                def fire(j, c2):
                    pltpu.async_copy(
                        table_hbm.at[idx_v.at[g * KG + j]],
                        rows_v.at[pl.ds((g * KG + j) * CHUNK, CHUNK)], sem)
                    return c2

                lax.fori_loop(0, KG, fire, 0)
                pltpu.make_async_copy(
                    out_hbm.at[pl.ds(0, KG * CHUNK)],
                    rows_v.at[pl.ds(g * KG * CHUNK, KG * CHUNK)], sem).wait()
                return carry

            lax.fori_loop(0, NCH // KG, gbody, 0)
            pltpu.sync_copy(rows_v, out_hbm.at[pl.ds(wid * EPW, EPW)])

        gather(src_pinned_hbm, net_hbm, gpinned_hbm)
        gather(src_near_hbm, node_hbm, gnear_hbm)

        pltpu.sync_copy(src_pins_hbm.at[wid], idx_v)

        def cgroup(g, carry):
            def fire(j, c2):
                pltpu.async_copy(ones_v, acc.at[idx_v.at[g * KG + j]], sem,
                                 add=True)
                return c2

            lax.fori_loop(0, KG, fire, 0)
            pltpu.make_async_copy(
                zeros_hbm.at[pl.ds(0, KG * CHUNK)], cnt_v, sem).wait()
            return carry

        lax.fori_loop(0, NCH // KG, cgroup, 0)
        plsc.subcore_barrier()
        pltpu.sync_copy(acc.at[pl.ds(sid * RPT, RPT)],
                        deg_hbm.at[cid, pl.ds(sid * RPT, RPT)])

    return k(net_feat, node_feat, src_pinned, src_near, src_pins, zeros32,
             ones32)


def _sc_agg(x32, msg_p, msg_n, src_pins, dst_pins, dst_pinned, dst_near,
            zeros32):
    mesh = plsc.VectorSubcoreMesh(core_axis_name="c", subcore_axis_name="s")

    @functools.partial(
        pl.kernel,
        out_type=[jax.ShapeDtypeStruct((NC, 3, NP, W), jnp.float32)],
        mesh=mesh,
        scratch_types=[
            pltpu.VMEM((NCH, CHUNK), jnp.int32),
            pltpu.VMEM((NCH, CHUNK), jnp.int32),
            pltpu.VMEM((HROWS, W), jnp.float32),
            pltpu.VMEM_SHARED((NP, W), jnp.float32),
            pltpu.SemaphoreType.DMA,
        ],
        compiler_params=pltpu.CompilerParams(use_tc_tiling_on_sc=False),
    )
    def k(x32_hbm, msg_p_hbm, msg_n_hbm, src_pins_hbm, dst_pins_hbm,
          dst_pinned_hbm, dst_near_hbm, zeros_hbm, out_hbm,
          idx_s, idx_d, rows_v, acc, sem):
        cid = lax.axis_index("c")
        sid = lax.axis_index("s")
        wid = sid * NC + cid

        def zero_acc():
            pltpu.sync_copy(zeros_hbm.at[pl.ds(sid * RPT, RPT)],
                            acc.at[pl.ds(sid * RPT, RPT)])
            plsc.subcore_barrier()

        def flush_acc(r):
            plsc.subcore_barrier()
            pltpu.sync_copy(acc.at[pl.ds(sid * RPT, RPT)],
                            out_hbm.at[cid, r, pl.ds(sid * RPT, RPT)])

        def drain(n_rows):
            pltpu.make_async_copy(x32_hbm.at[pl.ds(0, n_rows)],
                                  rows_v.at[pl.ds(0, n_rows)], sem).wait()

        zero_acc()
        pltpu.sync_copy(src_pins_hbm.at[wid], idx_s)
        pltpu.sync_copy(dst_pins_hbm.at[wid], idx_d)

        def pphase(q, carry):
            def fire_g(j, c2):
                pltpu.async_copy(x32_hbm.at[idx_s.at[q * KG + j]],
                                 rows_v.at[pl.ds(j * CHUNK, CHUNK)], sem)
                return c2

            lax.fori_loop(0, KG, fire_g, 0)
            drain(KG * CHUNK)

            def fire_s(j, c2):
                pltpu.async_copy(rows_v.at[pl.ds(j * CHUNK, CHUNK)],
                                 acc.at[idx_d.at[q * KG + j]], sem, add=True)
                return c2

            lax.fori_loop(0, KG, fire_s, 0)
            drain(KG * CHUNK)
            return carry

        lax.fori_loop(0, NCH // KG, pphase, 0)
        flush_acc(0)

        def scat(msg_hbm):
            def hbody(h, carry):
                pltpu.sync_copy(
                    msg_hbm.at[pl.ds(wid * EPW + h * HROWS, HROWS)], rows_v)

                def fire_s(j, c2):
                    pltpu.async_copy(rows_v.at[pl.ds(j * CHUNK, CHUNK)],
                                     acc.at[idx_d.at[h * HCH + j]], sem,
                                     add=True)
                    return c2

                lax.fori_loop(0, HCH, fire_s, 0)
                drain(HROWS)
                return carry

            lax.fori_loop(0, EPW // HROWS, hbody, 0)

        zero_acc()
        pltpu.sync_copy(dst_pinned_hbm.at[wid], idx_d)
        scat(msg_p_hbm)
        flush_acc(1)

        zero_acc()
        pltpu.sync_copy(dst_near_hbm.at[wid], idx_d)
        scat(msg_n_hbm)
        flush_acc(2)

    (out,) = k(x32, msg_p, msg_n, src_pins, dst_pins, dst_pinned, dst_near,
               zeros32)
    return out


def _tc_scale(nf_pad, deg_parts):
    def body(nf_ref, d_ref, o_ref):
        deg8 = d_ref[0] + d_ref[1]
        deg = jnp.concatenate([deg8, deg8], axis=1)
        x16 = nf_ref[...] * lax.rsqrt(jnp.maximum(deg, 1.0))
        o_ref[...] = jnp.concatenate(
            [x16, jnp.ones((NP, 8), jnp.float32)], axis=1)

    return pl.pallas_call(
        body, out_shape=jax.ShapeDtypeStruct((NP, W), jnp.float32),
    )(nf_pad, deg_parts)


def _tc_msg(g_p, g_n, ef_p, ef_n, w_p, b_p, w_n, b_n, r_c, t_c, blk):

    nblk = E // blk

    def body(gp_ref, gn_ref, efp_ref, efn_ref, wp_ref, bp_ref, wn_ref,
             bn_ref, r_ref, t_ref, op_ref, on_ref):
        ones = (lax.broadcasted_iota(jnp.int32, (blk, W), 1) >= 16).astype(
            jnp.float32)

        def msg(g_r, ef_r, w_r, b_r, o_r):
            w_e = jnp.dot(ef_r[...], w_r[...],
                          preferred_element_type=jnp.float32) + b_r[...]
            fx = jnp.dot(g_r[...], r_ref[...],
                         preferred_element_type=jnp.float32)
            m = jnp.dot(w_e * fx, t_ref[...],
                        preferred_element_type=jnp.float32)
            o_r[...] = m + ones

        msg(gp_ref, efp_ref, wp_ref, bp_ref, op_ref)
        msg(gn_ref, efn_ref, wn_ref, bn_ref, on_ref)

    edge_spec = pl.BlockSpec((blk, 16), lambda i: (i, 0))
    const_specs = [
        pl.BlockSpec((16, 256), lambda i: (0, 0)),
        pl.BlockSpec((1, 256), lambda i: (0, 0)),
        pl.BlockSpec((16, 256), lambda i: (0, 0)),
        pl.BlockSpec((1, 256), lambda i: (0, 0)),
        pl.BlockSpec((16, 256), lambda i: (0, 0)),
        pl.BlockSpec((256, W), lambda i: (0, 0)),
    ]
    out_spec = pl.BlockSpec((blk, W), lambda i: (i, 0))
    return pl.pallas_call(
        body,
        grid=(nblk,),
        in_specs=[edge_spec] * 4 + const_specs,
        out_specs=[out_spec, out_spec],
        out_shape=[jax.ShapeDtypeStruct((EP, W), jnp.float32),
                   jax.ShapeDtypeStruct((EP, W), jnp.float32)],
    )(g_p, g_n, ef_p, ef_n, w_p, b_p, w_n, b_n, r_c, t_c)


def _tc_final(parts, w_gc, b_gc, b_pinned, b_near):
    def body(p_ref, w_ref, bg_ref, bp_ref, bn_ref, node_ref, net_ref):
        def cnt16(r):
            c8 = p_ref[0, r, :, 16:W] + p_ref[1, r, :, 16:W]
            return jnp.concatenate([c8, c8], axis=1)

        agg = p_ref[0, 0, :, :16] + p_ref[1, 0, :, :16]
        rst = agg * lax.rsqrt(jnp.maximum(cnt16(0), 1.0))
        net_ref[...] = jnp.dot(rst, w_ref[...],
                               preferred_element_type=jnp.float32) + bg_ref[...]
        s1 = p_ref[0, 1, :, :16] + p_ref[1, 1, :, :16]
        o1 = s1 / jnp.maximum(cnt16(1), 1.0) + bp_ref[...]
        s2 = p_ref[0, 2, :, :16] + p_ref[1, 2, :, :16]
        o2 = s2 / jnp.maximum(cnt16(2), 1.0) + bn_ref[...]
        node_ref[...] = jnp.maximum(o1, o2)

    blk = NP // 8
    out_spec = pl.BlockSpec((blk, 16), lambda i: (i, 0))
    return pl.pallas_call(
        body,
        grid=(8,),
        in_specs=[
            pl.BlockSpec((NC, 3, blk, W), lambda i: (0, 0, i, 0)),
            pl.BlockSpec((16, 16), lambda i: (0, 0)),
            pl.BlockSpec((1, 16), lambda i: (0, 0)),
            pl.BlockSpec((1, 16), lambda i: (0, 0)),
            pl.BlockSpec((1, 16), lambda i: (0, 0)),
        ],
        out_specs=[out_spec, out_spec],
        out_shape=[jax.ShapeDtypeStruct((NP, 16), jnp.float32),
                   jax.ShapeDtypeStruct((NP, 16), jnp.float32)],
    )(parts, w_gc, b_gc, b_pinned, b_near)


def kernel(node_feat, net_feat, pin_feat, edge_feat, pins_edge_index,
           pinned_edge_index, near_edge_index, w_gc, b_gc, w_topo, b_topo,
           w_geom, b_geom, b_pinned, b_near):
    f32 = jnp.float32

    def prep_idx(a, fill):
        pad = jnp.full((EP - E,), fill, jnp.int32)
        return jnp.concatenate([a.astype(jnp.int32), pad]).reshape(
            NW, NCH, CHUNK)

    def pad_rows(a, n):
        return jnp.concatenate(
            [a, jnp.zeros((n - a.shape[0], a.shape[1]), a.dtype)])

    src_pins = prep_idx(pins_edge_index[0], SENT)
    dst_pins = prep_idx(pins_edge_index[1], SENT)
    src_pinned = prep_idx(pinned_edge_index[0], 0)
    dst_pinned = prep_idx(pinned_edge_index[1], SENT)
    src_near = prep_idx(near_edge_index[0], 0)
    dst_near = prep_idx(near_edge_index[1], SENT)

    zeros24 = jnp.zeros((NP, W), f32)
    zeros8 = jnp.zeros((NP, 8), f32)
    ones8 = jnp.ones((CHUNK, 8), f32)
    nf_pad = pad_rows(node_feat, NP)
    r_c = jnp.asarray(_R_NP)
    t_c = jnp.asarray(_T24_NP)

    gpinned, gnear, deg_parts = _sc_front(
        net_feat, node_feat, src_pinned, src_near, src_pins, zeros8, ones8)
    x32 = _tc_scale(nf_pad, deg_parts)
    msg_p, msg_n = _tc_msg(gpinned, gnear, pin_feat, edge_feat, w_topo,
                           b_topo.reshape(1, 256), w_geom,
                           b_geom.reshape(1, 256), r_c, t_c, 2000)
    parts = _sc_agg(x32, msg_p, msg_n, src_pins, dst_pins, dst_pinned,
                    dst_near, zeros24)
    node_out, net_out = _tc_final(parts, w_gc, b_gc.reshape(1, 16),
                                  b_pinned.reshape(1, 16),
                                  b_near.reshape(1, 16))
    return node_out[:N], net_out[:N]


# speedup vs baseline: 1.1138x; 1.0374x over previous
import functools

import jax
import jax.numpy as jnp
import numpy as np
from jax import lax
from jax.experimental import pallas as pl
from jax.experimental.pallas import tpu as pltpu
from jax.experimental.pallas import tpu_sc as plsc

N = 10000
SENT = N
NP = 10112
E = 160000
EP = 163840
NC = 2
NS = 16
NW = NC * NS
CHUNK = 128
EPW = EP // NW
NCH = EPW // CHUNK
RPT = NP // NS
HROWS = 2560
HCH = HROWS // CHUNK
KG = 10

W = 24
_R_NP = np.kron(np.eye(16, dtype=np.float32), np.ones((1, 16), np.float32))
_T24_NP = np.concatenate(
    [np.kron(np.ones((16, 1), np.float32), np.eye(16, dtype=np.float32)),
     np.zeros((256, 8), np.float32)], axis=1)


def _sc_front(net_feat, node_feat, src_pinned, src_near, src_pins, zeros32,
              ones32):
    mesh = plsc.VectorSubcoreMesh(core_axis_name="c", subcore_axis_name="s")

    @functools.partial(
        pl.kernel,
        out_type=[
            jax.ShapeDtypeStruct((EP, 16), jnp.float32),
            jax.ShapeDtypeStruct((EP, 16), jnp.float32),
            jax.ShapeDtypeStruct((NC, NP, 8), jnp.float32),
        ],
        mesh=mesh,
        scratch_types=[
            pltpu.VMEM((NCH, CHUNK), jnp.int32),
            pltpu.VMEM((EPW, 16), jnp.float32),
            pltpu.VMEM((CHUNK, 8), jnp.float32),
            pltpu.VMEM((KG * CHUNK, 8), jnp.float32),
            pltpu.VMEM_SHARED((NP, 8), jnp.float32),
            pltpu.SemaphoreType.DMA,
        ],
        compiler_params=pltpu.CompilerParams(use_tc_tiling_on_sc=False),
    )
    def k(net_hbm, node_hbm, src_pinned_hbm, src_near_hbm, src_pins_hbm,
          zeros_hbm, ones_hbm, gpinned_hbm, gnear_hbm, deg_hbm,
          idx_v, rows_v, ones_v, cnt_v, acc, sem):
        cid = lax.axis_index("c")
        sid = lax.axis_index("s")
        wid = sid * NC + cid
        pltpu.sync_copy(zeros_hbm.at[pl.ds(sid * RPT, RPT)],
                        acc.at[pl.ds(sid * RPT, RPT)])
        pltpu.sync_copy(ones_hbm, ones_v)
        plsc.subcore_barrier()

        def gather(src_hbm, table_hbm, out_hbm):
            pltpu.sync_copy(src_hbm.at[wid], idx_v)

            def gbody(g, carry):
                def fire(j, c2):
                    pltpu.async_copy(
                        table_hbm.at[idx_v.at[g * KG + j]],
                        rows_v.at[pl.ds((g * KG + j) * CHUNK, CHUNK)], sem)
                    return c2

                lax.fori_loop(0, KG, fire, 0)
                pltpu.make_async_copy(
                    out_hbm.at[pl.ds(0, KG * CHUNK)],
                    rows_v.at[pl.ds(g * KG * CHUNK, KG * CHUNK)], sem).wait()
                return carry

            lax.fori_loop(0, NCH // KG, gbody, 0)
            pltpu.sync_copy(rows_v, out_hbm.at[pl.ds(wid * EPW, EPW)])

        gather(src_pinned_hbm, net_hbm, gpinned_hbm)
        gather(src_near_hbm, node_hbm, gnear_hbm)

        pltpu.sync_copy(src_pins_hbm.at[wid], idx_v)

        def cgroup(g, carry):
            def fire(j, c2):
                pltpu.async_copy(ones_v, acc.at[idx_v.at[g * KG + j]], sem,
                                 add=True)
                return c2

            lax.fori_loop(0, KG, fire, 0)
            pltpu.make_async_copy(
                zeros_hbm.at[pl.ds(0, KG * CHUNK)], cnt_v, sem).wait()
            return carry

        lax.fori_loop(0, NCH // KG, cgroup, 0)
        plsc.subcore_barrier()
        pltpu.sync_copy(acc.at[pl.ds(sid * RPT, RPT)],
                        deg_hbm.at[cid, pl.ds(sid * RPT, RPT)])

    return k(net_feat, node_feat, src_pinned, src_near, src_pins, zeros32,
             ones32)


def _sc_agg(x32, msg_p, msg_n, src_pins, dst_pins, dst_pinned, dst_near,
            zeros32):
    mesh = plsc.VectorSubcoreMesh(core_axis_name="c", subcore_axis_name="s")

    @functools.partial(
        pl.kernel,
        out_type=[jax.ShapeDtypeStruct((NC, 3, NP, W), jnp.float32)],
        mesh=mesh,
        scratch_types=[
            pltpu.VMEM((NCH, CHUNK), jnp.int32),
            pltpu.VMEM((NCH, CHUNK), jnp.int32),
            pltpu.VMEM((HROWS, W), jnp.float32),
            pltpu.VMEM_SHARED((NP, W), jnp.float32),
            pltpu.SemaphoreType.DMA,
        ],
        compiler_params=pltpu.CompilerParams(use_tc_tiling_on_sc=False),
    )
    def k(x32_hbm, msg_p_hbm, msg_n_hbm, src_pins_hbm, dst_pins_hbm,
          dst_pinned_hbm, dst_near_hbm, zeros_hbm, out_hbm,
          idx_s, idx_d, rows_v, acc, sem):
        cid = lax.axis_index("c")
        sid = lax.axis_index("s")
        wid = sid * NC + cid

        def zero_acc():
            pltpu.sync_copy(zeros_hbm.at[pl.ds(sid * RPT, RPT)],
                            acc.at[pl.ds(sid * RPT, RPT)])
            plsc.subcore_barrier()

        def flush_acc(r):
            plsc.subcore_barrier()
            pltpu.sync_copy(acc.at[pl.ds(sid * RPT, RPT)],
                            out_hbm.at[cid, r, pl.ds(sid * RPT, RPT)])

        def drain(n_rows):
            pltpu.make_async_copy(x32_hbm.at[pl.ds(0, n_rows)],
                                  rows_v.at[pl.ds(0, n_rows)], sem).wait()

        zero_acc()
        pltpu.sync_copy(src_pins_hbm.at[wid], idx_s)
        pltpu.sync_copy(dst_pins_hbm.at[wid], idx_d)

        def pphase(q, carry):
            def fire_g(j, c2):
                pltpu.async_copy(x32_hbm.at[idx_s.at[q * KG + j]],
                                 rows_v.at[pl.ds(j * CHUNK, CHUNK)], sem)
                return c2

            lax.fori_loop(0, KG, fire_g, 0)
            drain(KG * CHUNK)

            def fire_s(j, c2):
                pltpu.async_copy(rows_v.at[pl.ds(j * CHUNK, CHUNK)],
                                 acc.at[idx_d.at[q * KG + j]], sem, add=True)
                return c2

            lax.fori_loop(0, KG, fire_s, 0)
            drain(KG * CHUNK)
            return carry

        lax.fori_loop(0, NCH // KG, pphase, 0)
        flush_acc(0)

        def scat(msg_hbm):
            def hbody(h, carry):
                pltpu.sync_copy(
                    msg_hbm.at[pl.ds(wid * EPW + h * HROWS, HROWS)], rows_v)

                def fire_s(j, c2):
                    pltpu.async_copy(rows_v.at[pl.ds(j * CHUNK, CHUNK)],
                                     acc.at[idx_d.at[h * HCH + j]], sem,
                                     add=True)
                    return c2

                lax.fori_loop(0, HCH, fire_s, 0)
                drain(HROWS)
                return carry

            lax.fori_loop(0, EPW // HROWS, hbody, 0)

        zero_acc()
        pltpu.sync_copy(dst_pinned_hbm.at[wid], idx_d)
        scat(msg_p_hbm)
        flush_acc(1)

        zero_acc()
        pltpu.sync_copy(dst_near_hbm.at[wid], idx_d)
        scat(msg_n_hbm)
        flush_acc(2)

    (out,) = k(x32, msg_p, msg_n, src_pins, dst_pins, dst_pinned, dst_near,
               zeros32)
    return out


def _tc_scale(nf_pad, deg_parts):
    def body(nf_ref, d_ref, o_ref):
        deg8 = d_ref[0] + d_ref[1]
        deg = jnp.concatenate([deg8, deg8], axis=1)
        x16 = nf_ref[...] * lax.rsqrt(jnp.maximum(deg, 1.0))
        o_ref[...] = jnp.concatenate(
            [x16, jnp.ones((NP, 8), jnp.float32)], axis=1)

    return pl.pallas_call(
        body, out_shape=jax.ShapeDtypeStruct((NP, W), jnp.float32),
    )(nf_pad, deg_parts)


def _tc_msg(g_p, g_n, ef_p, ef_n, w_p, b_p, w_n, b_n, r_c, t_c, blk):

    nblk = E // blk

    def body(gp_ref, gn_ref, efp_ref, efn_ref, wp_ref, bp_ref, wn_ref,
             bn_ref, r_ref, t_ref, op_ref, on_ref):
        ones = (lax.broadcasted_iota(jnp.int32, (blk, W), 1) >= 16).astype(
            jnp.float32)

        def msg(g_r, ef_r, w_r, b_r, o_r):
            w_e = jnp.dot(ef_r[...], w_r[...],
                          preferred_element_type=jnp.float32) + b_r[...]
            fx = jnp.dot(g_r[...], r_ref[...],
                         preferred_element_type=jnp.float32)
            m = jnp.dot(w_e * fx, t_ref[...],
                        preferred_element_type=jnp.float32)
            o_r[...] = m + ones

        msg(gp_ref, efp_ref, wp_ref, bp_ref, op_ref)
        msg(gn_ref, efn_ref, wn_ref, bn_ref, on_ref)

    edge_spec = pl.BlockSpec((blk, 16), lambda i: (i, 0))
    const_specs = [
        pl.BlockSpec((16, 256), lambda i: (0, 0)),
        pl.BlockSpec((1, 256), lambda i: (0, 0)),
        pl.BlockSpec((16, 256), lambda i: (0, 0)),
        pl.BlockSpec((1, 256), lambda i: (0, 0)),
        pl.BlockSpec((16, 256), lambda i: (0, 0)),
        pl.BlockSpec((256, W), lambda i: (0, 0)),
    ]
    out_spec = pl.BlockSpec((blk, W), lambda i: (i, 0))
    return pl.pallas_call(
        body,
        grid=(nblk,),
        in_specs=[edge_spec] * 4 + const_specs,
        out_specs=[out_spec, out_spec],
        out_shape=[jax.ShapeDtypeStruct((EP, W), jnp.float32),
                   jax.ShapeDtypeStruct((EP, W), jnp.float32)],
    )(g_p, g_n, ef_p, ef_n, w_p, b_p, w_n, b_n, r_c, t_c)


def _tc_final(parts, w_gc, b_gc, b_pinned, b_near):
    def body(p_ref, w_ref, bg_ref, bp_ref, bn_ref, node_ref, net_ref):
        def cnt16(r):
            c8 = p_ref[0, r, :, 16:W] + p_ref[1, r, :, 16:W]
            return jnp.concatenate([c8, c8], axis=1)

        agg = p_ref[0, 0, :, :16] + p_ref[1, 0, :, :16]
        rst = agg * lax.rsqrt(jnp.maximum(cnt16(0), 1.0))
        net_ref[...] = jnp.dot(rst, w_ref[...],
                               preferred_element_type=jnp.float32) + bg_ref[...]
        s1 = p_ref[0, 1, :, :16] + p_ref[1, 1, :, :16]
        o1 = s1 / jnp.maximum(cnt16(1), 1.0) + bp_ref[...]
        s2 = p_ref[0, 2, :, :16] + p_ref[1, 2, :, :16]
        o2 = s2 / jnp.maximum(cnt16(2), 1.0) + bn_ref[...]
        node_ref[...] = jnp.maximum(o1, o2)

    blk = NP // 8
    out_spec = pl.BlockSpec((blk, 16), lambda i: (i, 0))
    return pl.pallas_call(
        body,
        grid=(8,),
        in_specs=[
            pl.BlockSpec((NC, 3, blk, W), lambda i: (0, 0, i, 0)),
            pl.BlockSpec((16, 16), lambda i: (0, 0)),
            pl.BlockSpec((1, 16), lambda i: (0, 0)),
            pl.BlockSpec((1, 16), lambda i: (0, 0)),
            pl.BlockSpec((1, 16), lambda i: (0, 0)),
        ],
        out_specs=[out_spec, out_spec],
        out_shape=[jax.ShapeDtypeStruct((NP, 16), jnp.float32),
                   jax.ShapeDtypeStruct((NP, 16), jnp.float32)],
    )(parts, w_gc, b_gc, b_pinned, b_near)


def kernel(node_feat, net_feat, pin_feat, edge_feat, pins_edge_index,
           pinned_edge_index, near_edge_index, w_gc, b_gc, w_topo, b_topo,
           w_geom, b_geom, b_pinned, b_near):
    f32 = jnp.float32

    def prep_idx(a, fill):
        pad = jnp.full((EP - E,), fill, jnp.int32)
        return jnp.concatenate([a.astype(jnp.int32), pad]).reshape(
            NW, NCH, CHUNK)

    def pad_rows(a, n):
        return jnp.concatenate(
            [a, jnp.zeros((n - a.shape[0], a.shape[1]), a.dtype)])

    src_pins = prep_idx(pins_edge_index[0], SENT)
    dst_pins = prep_idx(pins_edge_index[1], SENT)
    src_pinned = prep_idx(pinned_edge_index[0], 0)
    dst_pinned = prep_idx(pinned_edge_index[1], SENT)
    src_near = prep_idx(near_edge_index[0], 0)
    dst_near = prep_idx(near_edge_index[1], SENT)

    zeros24 = jnp.zeros((NP, W), f32)
    zeros8 = jnp.zeros((NP, 8), f32)
    ones8 = jnp.ones((CHUNK, 8), f32)
    nf_pad = pad_rows(node_feat, NP)
    r_c = jnp.asarray(_R_NP)
    t_c = jnp.asarray(_T24_NP)

    gpinned, gnear, deg_parts = _sc_front(
        net_feat, node_feat, src_pinned, src_near, src_pins, zeros8, ones8)
    x32 = _tc_scale(nf_pad, deg_parts)
    msg_p, msg_n = _tc_msg(gpinned, gnear, pin_feat, edge_feat, w_topo,
                           b_topo.reshape(1, 256), w_geom,
                           b_geom.reshape(1, 256), r_c, t_c, 4000)
    parts = _sc_agg(x32, msg_p, msg_n, src_pins, dst_pins, dst_pinned,
                    dst_near, zeros24)
    node_out, net_out = _tc_final(parts, w_gc, b_gc.reshape(1, 16),
                                  b_pinned.reshape(1, 16),
                                  b_near.reshape(1, 16))
    return node_out[:N], net_out[:N]


# speedup vs baseline: 1.1284x; 1.0130x over previous
import functools

import jax
import jax.numpy as jnp
import numpy as np
from jax import lax
from jax.experimental import pallas as pl
from jax.experimental.pallas import tpu as pltpu
from jax.experimental.pallas import tpu_sc as plsc

N = 10000
SENT = N
NP = 10112
E = 160000
EP = 163840
NC = 2
NS = 16
NW = NC * NS
CHUNK = 128
EPW = EP // NW
NCH = EPW // CHUNK
RPT = NP // NS
HROWS = 2560
HCH = HROWS // CHUNK
KG = 10

W = 24
_R_NP = np.kron(np.eye(16, dtype=np.float32), np.ones((1, 16), np.float32))
_T24_NP = np.concatenate(
    [np.kron(np.ones((16, 1), np.float32), np.eye(16, dtype=np.float32)),
     np.zeros((256, 8), np.float32)], axis=1)


def _sc_front(net_feat, node_feat, src_pinned, src_near, src_pins, zeros32,
              ones32):
    mesh = plsc.VectorSubcoreMesh(core_axis_name="c", subcore_axis_name="s")

    @functools.partial(
        pl.kernel,
        out_type=[
            jax.ShapeDtypeStruct((EP, 16), jnp.float32),
            jax.ShapeDtypeStruct((EP, 16), jnp.float32),
            jax.ShapeDtypeStruct((NC, NP, 8), jnp.float32),
        ],
        mesh=mesh,
        scratch_types=[
            pltpu.VMEM((NCH, CHUNK), jnp.int32),
            pltpu.VMEM((EPW, 16), jnp.float32),
            pltpu.VMEM((CHUNK, 8), jnp.float32),
            pltpu.VMEM((KG * CHUNK, 8), jnp.float32),
            pltpu.VMEM_SHARED((NP, 8), jnp.float32),
            pltpu.SemaphoreType.DMA,
        ],
        compiler_params=pltpu.CompilerParams(use_tc_tiling_on_sc=False),
    )
    def k(net_hbm, node_hbm, src_pinned_hbm, src_near_hbm, src_pins_hbm,
          zeros_hbm, ones_hbm, gpinned_hbm, gnear_hbm, deg_hbm,
          idx_v, rows_v, ones_v, cnt_v, acc, sem):
        cid = lax.axis_index("c")
        sid = lax.axis_index("s")
        wid = sid * NC + cid
        pltpu.sync_copy(zeros_hbm.at[pl.ds(sid * RPT, RPT)],
                        acc.at[pl.ds(sid * RPT, RPT)])
        pltpu.sync_copy(ones_hbm, ones_v)
        plsc.subcore_barrier()

        def gather(src_hbm, table_hbm, out_hbm):
            pltpu.sync_copy(src_hbm.at[wid], idx_v)

            def gbody(g, carry):
                def fire(j, c2):
                    pltpu.async_copy(
                        table_hbm.at[idx_v.at[g * KG + j]],
                        rows_v.at[pl.ds((g * KG + j) * CHUNK, CHUNK)], sem)
                    return c2

                lax.fori_loop(0, KG, fire, 0)
                pltpu.make_async_copy(
                    out_hbm.at[pl.ds(0, KG * CHUNK)],
                    rows_v.at[pl.ds(g * KG * CHUNK, KG * CHUNK)], sem).wait()
                return carry

            lax.fori_loop(0, NCH // KG, gbody, 0)
            pltpu.sync_copy(rows_v, out_hbm.at[pl.ds(wid * EPW, EPW)])

        gather(src_pinned_hbm, net_hbm, gpinned_hbm)
        gather(src_near_hbm, node_hbm, gnear_hbm)

        pltpu.sync_copy(src_pins_hbm.at[wid], idx_v)

        def cgroup(g, carry):
            def fire(j, c2):
                pltpu.async_copy(ones_v, acc.at[idx_v.at[g * KG + j]], sem,
                                 add=True)
                return c2

            lax.fori_loop(0, KG, fire, 0)
            pltpu.make_async_copy(
                zeros_hbm.at[pl.ds(0, KG * CHUNK)], cnt_v, sem).wait()
            return carry

        lax.fori_loop(0, NCH // KG, cgroup, 0)
        plsc.subcore_barrier()
        pltpu.sync_copy(acc.at[pl.ds(sid * RPT, RPT)],
                        deg_hbm.at[cid, pl.ds(sid * RPT, RPT)])

    return k(net_feat, node_feat, src_pinned, src_near, src_pins, zeros32,
             ones32)


def _sc_agg(x32, msg_p, msg_n, src_pins, dst_pins, dst_pinned, dst_near,
            zeros32):
    mesh = plsc.VectorSubcoreMesh(core_axis_name="c", subcore_axis_name="s")

    @functools.partial(
        pl.kernel,
        out_type=[jax.ShapeDtypeStruct((NC, 3, NP, W), jnp.float32)],
        mesh=mesh,
        scratch_types=[
            pltpu.VMEM((NCH, CHUNK), jnp.int32),
            pltpu.VMEM((NCH, CHUNK), jnp.int32),
            pltpu.VMEM((HROWS, W), jnp.float32),
            pltpu.VMEM_SHARED((NP, W), jnp.float32),
            pltpu.SemaphoreType.DMA,
        ],
        compiler_params=pltpu.CompilerParams(use_tc_tiling_on_sc=False),
    )
    def k(x32_hbm, msg_p_hbm, msg_n_hbm, src_pins_hbm, dst_pins_hbm,
          dst_pinned_hbm, dst_near_hbm, zeros_hbm, out_hbm,
          idx_s, idx_d, rows_v, acc, sem):
        cid = lax.axis_index("c")
        sid = lax.axis_index("s")
        wid = sid * NC + cid

        def zero_acc():
            pltpu.sync_copy(zeros_hbm.at[pl.ds(sid * RPT, RPT)],
                            acc.at[pl.ds(sid * RPT, RPT)])
            plsc.subcore_barrier()

        def flush_acc(r):
            plsc.subcore_barrier()
            pltpu.sync_copy(acc.at[pl.ds(sid * RPT, RPT)],
                            out_hbm.at[cid, r, pl.ds(sid * RPT, RPT)])

        def drain(n_rows):
            pltpu.make_async_copy(x32_hbm.at[pl.ds(0, n_rows)],
                                  rows_v.at[pl.ds(0, n_rows)], sem).wait()

        zero_acc()
        pltpu.sync_copy(src_pins_hbm.at[wid], idx_s)
        pltpu.sync_copy(dst_pins_hbm.at[wid], idx_d)

        def pphase(q, carry):
            def fire_g(j, c2):
                pltpu.async_copy(x32_hbm.at[idx_s.at[q * KG + j]],
                                 rows_v.at[pl.ds(j * CHUNK, CHUNK)], sem)
                return c2

            lax.fori_loop(0, KG, fire_g, 0)
            drain(KG * CHUNK)

            def fire_s(j, c2):
                pltpu.async_copy(rows_v.at[pl.ds(j * CHUNK, CHUNK)],
                                 acc.at[idx_d.at[q * KG + j]], sem, add=True)
                return c2

            lax.fori_loop(0, KG, fire_s, 0)
            drain(KG * CHUNK)
            return carry

        lax.fori_loop(0, NCH // KG, pphase, 0)
        flush_acc(0)

        def scat(msg_hbm):
            def hbody(h, carry):
                pltpu.sync_copy(
                    msg_hbm.at[pl.ds(wid * EPW + h * HROWS, HROWS)], rows_v)

                def fire_s(j, c2):
                    pltpu.async_copy(rows_v.at[pl.ds(j * CHUNK, CHUNK)],
                                     acc.at[idx_d.at[h * HCH + j]], sem,
                                     add=True)
                    return c2

                lax.fori_loop(0, HCH, fire_s, 0)
                drain(HROWS)
                return carry

            lax.fori_loop(0, EPW // HROWS, hbody, 0)

        zero_acc()
        pltpu.sync_copy(dst_pinned_hbm.at[wid], idx_d)
        scat(msg_p_hbm)
        flush_acc(1)

        zero_acc()
        pltpu.sync_copy(dst_near_hbm.at[wid], idx_d)
        scat(msg_n_hbm)
        flush_acc(2)

    (out,) = k(x32, msg_p, msg_n, src_pins, dst_pins, dst_pinned, dst_near,
               zeros32)
    return out


def _tc_scale(nf_pad, deg_parts):
    def body(nf_ref, d_ref, o_ref):
        deg8 = d_ref[0] + d_ref[1]
        deg = jnp.concatenate([deg8, deg8], axis=1)
        x16 = nf_ref[...] * lax.rsqrt(jnp.maximum(deg, 1.0))
        o_ref[...] = jnp.concatenate(
            [x16, jnp.ones((NP, 8), jnp.float32)], axis=1)

    return pl.pallas_call(
        body, out_shape=jax.ShapeDtypeStruct((NP, W), jnp.float32),
    )(nf_pad, deg_parts)


def _tc_msg(g_p, g_n, ef_p, ef_n, w_p, b_p, w_n, b_n, r_c, t_c, blk):

    nblk = E // blk

    def body(gp_ref, gn_ref, efp_ref, efn_ref, wp_ref, bp_ref, wn_ref,
             bn_ref, r_ref, t_ref, op_ref, on_ref):
        ones = (lax.broadcasted_iota(jnp.int32, (blk, W), 1) >= 16).astype(
            jnp.float32)

        def msg(g_r, ef_r, w_r, b_r, o_r):
            w_e = jnp.dot(ef_r[...], w_r[...],
                          preferred_element_type=jnp.float32) + b_r[...]
            fx = jnp.dot(g_r[...], r_ref[...],
                         preferred_element_type=jnp.float32)
            m = jnp.dot(w_e * fx, t_ref[...],
                        preferred_element_type=jnp.float32)
            o_r[...] = m + ones

        msg(gp_ref, efp_ref, wp_ref, bp_ref, op_ref)
        msg(gn_ref, efn_ref, wn_ref, bn_ref, on_ref)

    edge_spec = pl.BlockSpec((blk, 16), lambda i: (i, 0))
    const_specs = [
        pl.BlockSpec((16, 256), lambda i: (0, 0)),
        pl.BlockSpec((1, 256), lambda i: (0, 0)),
        pl.BlockSpec((16, 256), lambda i: (0, 0)),
        pl.BlockSpec((1, 256), lambda i: (0, 0)),
        pl.BlockSpec((16, 256), lambda i: (0, 0)),
        pl.BlockSpec((256, W), lambda i: (0, 0)),
    ]
    out_spec = pl.BlockSpec((blk, W), lambda i: (i, 0))
    return pl.pallas_call(
        body,
        grid=(nblk,),
        in_specs=[edge_spec] * 4 + const_specs,
        out_specs=[out_spec, out_spec],
        out_shape=[jax.ShapeDtypeStruct((EP, W), jnp.float32),
                   jax.ShapeDtypeStruct((EP, W), jnp.float32)],
    )(g_p, g_n, ef_p, ef_n, w_p, b_p, w_n, b_n, r_c, t_c)


def _tc_final(parts, w_gc, b_gc, b_pinned, b_near):
    def body(p_ref, w_ref, bg_ref, bp_ref, bn_ref, node_ref, net_ref):
        def cnt16(r):
            c8 = p_ref[0, r, :, 16:W] + p_ref[1, r, :, 16:W]
            return jnp.concatenate([c8, c8], axis=1)

        agg = p_ref[0, 0, :, :16] + p_ref[1, 0, :, :16]
        rst = agg * lax.rsqrt(jnp.maximum(cnt16(0), 1.0))
        net_ref[...] = jnp.dot(rst, w_ref[...],
                               preferred_element_type=jnp.float32) + bg_ref[...]
        s1 = p_ref[0, 1, :, :16] + p_ref[1, 1, :, :16]
        o1 = s1 / jnp.maximum(cnt16(1), 1.0) + bp_ref[...]
        s2 = p_ref[0, 2, :, :16] + p_ref[1, 2, :, :16]
        o2 = s2 / jnp.maximum(cnt16(2), 1.0) + bn_ref[...]
        node_ref[...] = jnp.maximum(o1, o2)

    blk = NP // 8
    out_spec = pl.BlockSpec((blk, 16), lambda i: (i, 0))
    return pl.pallas_call(
        body,
        grid=(8,),
        in_specs=[
            pl.BlockSpec((NC, 3, blk, W), lambda i: (0, 0, i, 0)),
            pl.BlockSpec((16, 16), lambda i: (0, 0)),
            pl.BlockSpec((1, 16), lambda i: (0, 0)),
            pl.BlockSpec((1, 16), lambda i: (0, 0)),
            pl.BlockSpec((1, 16), lambda i: (0, 0)),
        ],
        out_specs=[out_spec, out_spec],
        out_shape=[jax.ShapeDtypeStruct((NP, 16), jnp.float32),
                   jax.ShapeDtypeStruct((NP, 16), jnp.float32)],
    )(parts, w_gc, b_gc, b_pinned, b_near)


def kernel(node_feat, net_feat, pin_feat, edge_feat, pins_edge_index,
           pinned_edge_index, near_edge_index, w_gc, b_gc, w_topo, b_topo,
           w_geom, b_geom, b_pinned, b_near):
    f32 = jnp.float32

    def prep_idx(a, fill):
        pad = jnp.full((EP - E,), fill, jnp.int32)
        return jnp.concatenate([a.astype(jnp.int32), pad]).reshape(
            NW, NCH, CHUNK)

    def pad_rows(a, n):
        return jnp.concatenate(
            [a, jnp.zeros((n - a.shape[0], a.shape[1]), a.dtype)])

    src_pins = prep_idx(pins_edge_index[0], SENT)
    dst_pins = prep_idx(pins_edge_index[1], SENT)
    src_pinned = prep_idx(pinned_edge_index[0], 0)
    dst_pinned = prep_idx(pinned_edge_index[1], SENT)
    src_near = prep_idx(near_edge_index[0], 0)
    dst_near = prep_idx(near_edge_index[1], SENT)

    zeros24 = jnp.zeros((NP, W), f32)
    zeros8 = jnp.zeros((NP, 8), f32)
    ones8 = jnp.ones((CHUNK, 8), f32)
    nf_pad = pad_rows(node_feat, NP)
    r_c = jnp.asarray(_R_NP)
    t_c = jnp.asarray(_T24_NP)

    gpinned, gnear, deg_parts = _sc_front(
        net_feat, node_feat, src_pinned, src_near, src_pins, zeros8, ones8)
    x32 = _tc_scale(nf_pad, deg_parts)
    msg_p, msg_n = _tc_msg(gpinned, gnear, pin_feat, edge_feat, w_topo,
                           b_topo.reshape(1, 256), w_geom,
                           b_geom.reshape(1, 256), r_c, t_c, 8000)
    parts = _sc_agg(x32, msg_p, msg_n, src_pins, dst_pins, dst_pinned,
                    dst_near, zeros24)
    node_out, net_out = _tc_final(parts, w_gc, b_gc.reshape(1, 16),
                                  b_pinned.reshape(1, 16),
                                  b_near.reshape(1, 16))
    return node_out[:N], net_out[:N]


# speedup vs baseline: 1.1729x; 1.0395x over previous
import functools

import jax
import jax.numpy as jnp
import numpy as np
from jax import lax
from jax.experimental import pallas as pl
from jax.experimental.pallas import tpu as pltpu
from jax.experimental.pallas import tpu_sc as plsc

N = 10000
SENT = N
NP = 10112
E = 160000
EP = 163840
NC = 2
NS = 16
NW = NC * NS
CHUNK = 128
EPW = EP // NW
NCH = EPW // CHUNK
RPT = NP // NS
HROWS = 2560
HCH = HROWS // CHUNK
KG = 10

W = 24
_R_NP = np.kron(np.eye(16, dtype=np.float32), np.ones((1, 16), np.float32))
_T24_NP = np.concatenate(
    [np.kron(np.ones((16, 1), np.float32), np.eye(16, dtype=np.float32)),
     np.zeros((256, 8), np.float32)], axis=1)


def _sc_front(net_feat, node_feat, src_pinned, src_near, src_pins, zeros32,
              ones32):
    mesh = plsc.VectorSubcoreMesh(core_axis_name="c", subcore_axis_name="s")

    @functools.partial(
        pl.kernel,
        out_type=[
            jax.ShapeDtypeStruct((EP, 16), jnp.float32),
            jax.ShapeDtypeStruct((EP, 16), jnp.float32),
            jax.ShapeDtypeStruct((NC, NP, 8), jnp.float32),
        ],
        mesh=mesh,
        scratch_types=[
            pltpu.VMEM((NCH, CHUNK), jnp.int32),
            pltpu.VMEM((EPW, 16), jnp.float32),
            pltpu.VMEM((CHUNK, 8), jnp.float32),
            pltpu.VMEM((KG * CHUNK, 8), jnp.float32),
            pltpu.VMEM_SHARED((NP, 8), jnp.float32),
            pltpu.SemaphoreType.DMA,
        ],
        compiler_params=pltpu.CompilerParams(use_tc_tiling_on_sc=False),
    )
    def k(net_hbm, node_hbm, src_pinned_hbm, src_near_hbm, src_pins_hbm,
          zeros_hbm, ones_hbm, gpinned_hbm, gnear_hbm, deg_hbm,
          idx_v, rows_v, ones_v, cnt_v, acc, sem):
        cid = lax.axis_index("c")
        sid = lax.axis_index("s")
        wid = sid * NC + cid
        pltpu.sync_copy(zeros_hbm.at[pl.ds(sid * RPT, RPT)],
                        acc.at[pl.ds(sid * RPT, RPT)])
        pltpu.sync_copy(ones_hbm, ones_v)
        plsc.subcore_barrier()

        def gather(src_hbm, table_hbm, out_hbm):
            pltpu.sync_copy(src_hbm.at[wid], idx_v)

            def gbody(g, carry):
                def fire(j, c2):
                    pltpu.async_copy(
                        table_hbm.at[idx_v.at[g * KG + j]],
                        rows_v.at[pl.ds((g * KG + j) * CHUNK, CHUNK)], sem)
                    return c2

                lax.fori_loop(0, KG, fire, 0)
                pltpu.make_async_copy(
                    out_hbm.at[pl.ds(0, KG * CHUNK)],
                    rows_v.at[pl.ds(g * KG * CHUNK, KG * CHUNK)], sem).wait()
                return carry

            lax.fori_loop(0, NCH // KG, gbody, 0)
            pltpu.sync_copy(rows_v, out_hbm.at[pl.ds(wid * EPW, EPW)])

        gather(src_pinned_hbm, net_hbm, gpinned_hbm)
        gather(src_near_hbm, node_hbm, gnear_hbm)

        pltpu.sync_copy(src_pins_hbm.at[wid], idx_v)

        def cgroup(g, carry):
            def fire(j, c2):
                pltpu.async_copy(ones_v, acc.at[idx_v.at[g * KG + j]], sem,
                                 add=True)
                return c2

            lax.fori_loop(0, KG, fire, 0)
            pltpu.make_async_copy(
                zeros_hbm.at[pl.ds(0, KG * CHUNK)], cnt_v, sem).wait()
            return carry

        lax.fori_loop(0, NCH // KG, cgroup, 0)
        plsc.subcore_barrier()
        pltpu.sync_copy(acc.at[pl.ds(sid * RPT, RPT)],
                        deg_hbm.at[cid, pl.ds(sid * RPT, RPT)])

    return k(net_feat, node_feat, src_pinned, src_near, src_pins, zeros32,
             ones32)


def _sc_agg_pins(x32, src_pins, dst_pins, zeros32):
    mesh = plsc.VectorSubcoreMesh(core_axis_name="c", subcore_axis_name="s")

    @functools.partial(
        pl.kernel,
        out_type=[jax.ShapeDtypeStruct((NC, NP, W), jnp.float32)],
        mesh=mesh,
        scratch_types=[
            pltpu.VMEM((NCH, CHUNK), jnp.int32),
            pltpu.VMEM((NCH, CHUNK), jnp.int32),
            pltpu.VMEM((KG * CHUNK, W), jnp.float32),
            pltpu.VMEM_SHARED((NP, W), jnp.float32),
            pltpu.SemaphoreType.DMA,
        ],
        compiler_params=pltpu.CompilerParams(use_tc_tiling_on_sc=False),
    )
    def k(x32_hbm, src_pins_hbm, dst_pins_hbm, zeros_hbm, out_hbm,
          idx_s, idx_d, rows_v, acc, sem):
        cid = lax.axis_index("c")
        sid = lax.axis_index("s")
        wid = sid * NC + cid

        pltpu.sync_copy(zeros_hbm.at[pl.ds(sid * RPT, RPT)],
                        acc.at[pl.ds(sid * RPT, RPT)])
        plsc.subcore_barrier()
        pltpu.sync_copy(src_pins_hbm.at[wid], idx_s)
        pltpu.sync_copy(dst_pins_hbm.at[wid], idx_d)

        def drain(n_rows):
            pltpu.make_async_copy(x32_hbm.at[pl.ds(0, n_rows)],
                                  rows_v.at[pl.ds(0, n_rows)], sem).wait()

        def pphase(q, carry):
            def fire_g(j, c2):
                pltpu.async_copy(x32_hbm.at[idx_s.at[q * KG + j]],
                                 rows_v.at[pl.ds(j * CHUNK, CHUNK)], sem)
                return c2

            lax.fori_loop(0, KG, fire_g, 0)
            drain(KG * CHUNK)

            def fire_s(j, c2):
                pltpu.async_copy(rows_v.at[pl.ds(j * CHUNK, CHUNK)],
                                 acc.at[idx_d.at[q * KG + j]], sem, add=True)
                return c2

            lax.fori_loop(0, KG, fire_s, 0)
            drain(KG * CHUNK)
            return carry

        lax.fori_loop(0, NCH // KG, pphase, 0)
        plsc.subcore_barrier()
        pltpu.sync_copy(acc.at[pl.ds(sid * RPT, RPT)],
                        out_hbm.at[cid, pl.ds(sid * RPT, RPT)])

    (out,) = k(x32, src_pins, dst_pins, zeros32)
    return out


def _sc_agg_msg(msg_p, msg_n, dst_pinned, dst_near, zeros32):
    mesh = plsc.VectorSubcoreMesh(core_axis_name="c", subcore_axis_name="s")

    @functools.partial(
        pl.kernel,
        out_type=[jax.ShapeDtypeStruct((NC, 2, NP, W), jnp.float32)],
        mesh=mesh,
        scratch_types=[
            pltpu.VMEM((NCH, CHUNK), jnp.int32),
            pltpu.VMEM((HROWS, W), jnp.float32),
            pltpu.VMEM_SHARED((NP, W), jnp.float32),
            pltpu.SemaphoreType.DMA,
        ],
        compiler_params=pltpu.CompilerParams(use_tc_tiling_on_sc=False),
    )
    def k(msg_p_hbm, msg_n_hbm, dst_pinned_hbm, dst_near_hbm, zeros_hbm,
          out_hbm, idx_d, rows_v, acc, sem):
        cid = lax.axis_index("c")
        sid = lax.axis_index("s")
        wid = sid * NC + cid

        def zero_acc():
            pltpu.sync_copy(zeros_hbm.at[pl.ds(sid * RPT, RPT)],
                            acc.at[pl.ds(sid * RPT, RPT)])
            plsc.subcore_barrier()

        def flush_acc(r):
            plsc.subcore_barrier()
            pltpu.sync_copy(acc.at[pl.ds(sid * RPT, RPT)],
                            out_hbm.at[cid, r, pl.ds(sid * RPT, RPT)])

        def scat(msg_hbm):
            def hbody(h, carry):
                pltpu.sync_copy(
                    msg_hbm.at[pl.ds(wid * EPW + h * HROWS, HROWS)], rows_v)

                def fire_s(j, c2):
                    pltpu.async_copy(rows_v.at[pl.ds(j * CHUNK, CHUNK)],
                                     acc.at[idx_d.at[h * HCH + j]], sem,
                                     add=True)
                    return c2

                lax.fori_loop(0, HCH, fire_s, 0)
                pltpu.make_async_copy(msg_hbm.at[pl.ds(0, HROWS)],
                                      rows_v, sem).wait()
                return carry

            lax.fori_loop(0, EPW // HROWS, hbody, 0)

        zero_acc()
        pltpu.sync_copy(dst_pinned_hbm.at[wid], idx_d)
        scat(msg_p_hbm)
        flush_acc(0)

        zero_acc()
        pltpu.sync_copy(dst_near_hbm.at[wid], idx_d)
        scat(msg_n_hbm)
        flush_acc(1)

    (out,) = k(msg_p, msg_n, dst_pinned, dst_near, zeros32)
    return out


def _tc_scale(nf_pad, deg_parts):
    def body(nf_ref, d_ref, o_ref):
        deg8 = d_ref[0] + d_ref[1]
        deg = jnp.concatenate([deg8, deg8], axis=1)
        x16 = nf_ref[...] * lax.rsqrt(jnp.maximum(deg, 1.0))
        o_ref[...] = jnp.concatenate(
            [x16, jnp.ones((NP, 8), jnp.float32)], axis=1)

    return pl.pallas_call(
        body, out_shape=jax.ShapeDtypeStruct((NP, W), jnp.float32),
    )(nf_pad, deg_parts)


def _tc_msg(g_p, g_n, ef_p, ef_n, w_p, b_p, w_n, b_n, r_c, t_c, blk):

    nblk = E // blk

    def body(gp_ref, gn_ref, efp_ref, efn_ref, wp_ref, bp_ref, wn_ref,
             bn_ref, r_ref, t_ref, op_ref, on_ref):
        ones = (lax.broadcasted_iota(jnp.int32, (blk, W), 1) >= 16).astype(
            jnp.float32)

        def msg(g_r, ef_r, w_r, b_r, o_r):
            w_e = jnp.dot(ef_r[...], w_r[...],
                          preferred_element_type=jnp.float32) + b_r[...]
            fx = jnp.dot(g_r[...], r_ref[...],
                         preferred_element_type=jnp.float32)
            m = jnp.dot(w_e * fx, t_ref[...],
                        preferred_element_type=jnp.float32)
            o_r[...] = m + ones

        msg(gp_ref, efp_ref, wp_ref, bp_ref, op_ref)
        msg(gn_ref, efn_ref, wn_ref, bn_ref, on_ref)

    edge_spec = pl.BlockSpec((blk, 16), lambda i: (i, 0))
    const_specs = [
        pl.BlockSpec((16, 256), lambda i: (0, 0)),
        pl.BlockSpec((1, 256), lambda i: (0, 0)),
        pl.BlockSpec((16, 256), lambda i: (0, 0)),
        pl.BlockSpec((1, 256), lambda i: (0, 0)),
        pl.BlockSpec((16, 256), lambda i: (0, 0)),
        pl.BlockSpec((256, W), lambda i: (0, 0)),
    ]
    out_spec = pl.BlockSpec((blk, W), lambda i: (i, 0))
    return pl.pallas_call(
        body,
        grid=(nblk,),
        in_specs=[edge_spec] * 4 + const_specs,
        out_specs=[out_spec, out_spec],
        out_shape=[jax.ShapeDtypeStruct((EP, W), jnp.float32),
                   jax.ShapeDtypeStruct((EP, W), jnp.float32)],
    )(g_p, g_n, ef_p, ef_n, w_p, b_p, w_n, b_n, r_c, t_c)


def _tc_final(parts_pins, parts_msg, w_gc, b_gc, b_pinned, b_near):
    def body(pp_ref, pm_ref, w_ref, bg_ref, bp_ref, bn_ref, node_ref,
             net_ref):
        def cnt16(c8):
            return jnp.concatenate([c8, c8], axis=1)

        agg = pp_ref[0, :, :16] + pp_ref[1, :, :16]
        c0 = cnt16(pp_ref[0, :, 16:W] + pp_ref[1, :, 16:W])
        rst = agg * lax.rsqrt(jnp.maximum(c0, 1.0))
        net_ref[...] = jnp.dot(rst, w_ref[...],
                               preferred_element_type=jnp.float32) + bg_ref[...]
        s1 = pm_ref[0, 0, :, :16] + pm_ref[1, 0, :, :16]
        c1 = cnt16(pm_ref[0, 0, :, 16:W] + pm_ref[1, 0, :, 16:W])
        o1 = s1 / jnp.maximum(c1, 1.0) + bp_ref[...]
        s2 = pm_ref[0, 1, :, :16] + pm_ref[1, 1, :, :16]
        c2 = cnt16(pm_ref[0, 1, :, 16:W] + pm_ref[1, 1, :, 16:W])
        o2 = s2 / jnp.maximum(c2, 1.0) + bn_ref[...]
        node_ref[...] = jnp.maximum(o1, o2)

    blk = NP // 8
    out_spec = pl.BlockSpec((blk, 16), lambda i: (i, 0))
    return pl.pallas_call(
        body,
        grid=(8,),
        in_specs=[
            pl.BlockSpec((NC, blk, W), lambda i: (0, i, 0)),
            pl.BlockSpec((NC, 2, blk, W), lambda i: (0, 0, i, 0)),
            pl.BlockSpec((16, 16), lambda i: (0, 0)),
            pl.BlockSpec((1, 16), lambda i: (0, 0)),
            pl.BlockSpec((1, 16), lambda i: (0, 0)),
            pl.BlockSpec((1, 16), lambda i: (0, 0)),
        ],
        out_specs=[out_spec, out_spec],
        out_shape=[jax.ShapeDtypeStruct((NP, 16), jnp.float32),
                   jax.ShapeDtypeStruct((NP, 16), jnp.float32)],
    )(parts_pins, parts_msg, w_gc, b_gc, b_pinned, b_near)


def kernel(node_feat, net_feat, pin_feat, edge_feat, pins_edge_index,
           pinned_edge_index, near_edge_index, w_gc, b_gc, w_topo, b_topo,
           w_geom, b_geom, b_pinned, b_near):
    f32 = jnp.float32

    def prep_idx(a, fill):
        pad = jnp.full((EP - E,), fill, jnp.int32)
        return jnp.concatenate([a.astype(jnp.int32), pad]).reshape(
            NW, NCH, CHUNK)

    def pad_rows(a, n):
        return jnp.concatenate(
            [a, jnp.zeros((n - a.shape[0], a.shape[1]), a.dtype)])

    src_pins = prep_idx(pins_edge_index[0], SENT)
    dst_pins = prep_idx(pins_edge_index[1], SENT)
    src_pinned = prep_idx(pinned_edge_index[0], 0)
    dst_pinned = prep_idx(pinned_edge_index[1], SENT)
    src_near = prep_idx(near_edge_index[0], 0)
    dst_near = prep_idx(near_edge_index[1], SENT)

    zeros24 = jnp.zeros((NP, W), f32)
    zeros8 = jnp.zeros((NP, 8), f32)
    ones8 = jnp.ones((CHUNK, 8), f32)
    nf_pad = pad_rows(node_feat, NP)
    r_c = jnp.asarray(_R_NP)
    t_c = jnp.asarray(_T24_NP)

    gpinned, gnear, deg_parts = _sc_front(
        net_feat, node_feat, src_pinned, src_near, src_pins, zeros8, ones8)
    x32 = _tc_scale(nf_pad, deg_parts)
    msg_p, msg_n = _tc_msg(gpinned, gnear, pin_feat, edge_feat, w_topo,
                           b_topo.reshape(1, 256), w_geom,
                           b_geom.reshape(1, 256), r_c, t_c, 8000)
    parts_pins = _sc_agg_pins(x32, src_pins, dst_pins, zeros24)
    parts_msg = _sc_agg_msg(msg_p, msg_n, dst_pinned, dst_near, zeros24)
    node_out, net_out = _tc_final(parts_pins, parts_msg, w_gc,
                                  b_gc.reshape(1, 16),
                                  b_pinned.reshape(1, 16),
                                  b_near.reshape(1, 16))
    return node_out[:N], net_out[:N]


# speedup vs baseline: 1.1866x; 1.0117x over previous
import functools

import jax
import jax.numpy as jnp
import numpy as np
from jax import lax
from jax.experimental import pallas as pl
from jax.experimental.pallas import tpu as pltpu
from jax.experimental.pallas import tpu_sc as plsc

N = 10000
SENT = N
NP = 10112
E = 160000
EP = 163840
NC = 2
NS = 16
NW = NC * NS
CHUNK = 128
EPW = EP // NW
NCH = EPW // CHUNK
RPT = NP // NS
HROWS = 2560
HCH = HROWS // CHUNK
KG = 10

W = 24
_R_NP = np.kron(np.eye(16, dtype=np.float32), np.ones((1, 16), np.float32))
_T24_NP = np.concatenate(
    [np.kron(np.ones((16, 1), np.float32), np.eye(16, dtype=np.float32)),
     np.zeros((256, 8), np.float32)], axis=1)


def _sc_deg(src_pins, zeros8, ones8):
    mesh = plsc.VectorSubcoreMesh(core_axis_name="c", subcore_axis_name="s")

    @functools.partial(
        pl.kernel,
        out_type=[jax.ShapeDtypeStruct((NC, NP, 8), jnp.float32)],
        mesh=mesh,
        scratch_types=[
            pltpu.VMEM((NCH, CHUNK), jnp.int32),
            pltpu.VMEM((CHUNK, 8), jnp.float32),
            pltpu.VMEM((KG * CHUNK, 8), jnp.float32),
            pltpu.VMEM_SHARED((NP, 8), jnp.float32),
            pltpu.SemaphoreType.DMA,
        ],
        compiler_params=pltpu.CompilerParams(use_tc_tiling_on_sc=False),
    )
    def k(src_pins_hbm, zeros_hbm, ones_hbm, deg_hbm,
          idx_v, ones_v, cnt_v, acc, sem):
        cid = lax.axis_index("c")
        sid = lax.axis_index("s")
        wid = sid * NC + cid
        pltpu.sync_copy(zeros_hbm.at[pl.ds(sid * RPT, RPT)],
                        acc.at[pl.ds(sid * RPT, RPT)])
        pltpu.sync_copy(ones_hbm, ones_v)
        plsc.subcore_barrier()
        pltpu.sync_copy(src_pins_hbm.at[wid], idx_v)

        def cgroup(g, carry):
            def fire(j, c2):
                pltpu.async_copy(ones_v, acc.at[idx_v.at[g * KG + j]], sem,
                                 add=True)
                return c2

            lax.fori_loop(0, KG, fire, 0)
            pltpu.make_async_copy(
                zeros_hbm.at[pl.ds(0, KG * CHUNK)], cnt_v, sem).wait()
            return carry

        lax.fori_loop(0, NCH // KG, cgroup, 0)
        plsc.subcore_barrier()
        pltpu.sync_copy(acc.at[pl.ds(sid * RPT, RPT)],
                        deg_hbm.at[cid, pl.ds(sid * RPT, RPT)])

    (out,) = k(src_pins, zeros8, ones8)
    return out


def _sc_gather(net_feat, node_feat, src_pinned, src_near):
    mesh = plsc.VectorSubcoreMesh(core_axis_name="c", subcore_axis_name="s")

    @functools.partial(
        pl.kernel,
        out_type=[
            jax.ShapeDtypeStruct((EP, 16), jnp.float32),
            jax.ShapeDtypeStruct((EP, 16), jnp.float32),
        ],
        mesh=mesh,
        scratch_types=[
            pltpu.VMEM((NCH, CHUNK), jnp.int32),
            pltpu.VMEM((EPW, 16), jnp.float32),
            pltpu.SemaphoreType.DMA,
        ],
        compiler_params=pltpu.CompilerParams(use_tc_tiling_on_sc=False),
    )
    def k(net_hbm, node_hbm, src_pinned_hbm, src_near_hbm,
          gpinned_hbm, gnear_hbm, idx_v, rows_v, sem):
        cid = lax.axis_index("c")
        sid = lax.axis_index("s")
        wid = sid * NC + cid

        def gather(src_hbm, table_hbm, out_hbm):
            pltpu.sync_copy(src_hbm.at[wid], idx_v)

            def gbody(g, carry):
                def fire(j, c2):
                    pltpu.async_copy(
                        table_hbm.at[idx_v.at[g * KG + j]],
                        rows_v.at[pl.ds((g * KG + j) * CHUNK, CHUNK)], sem)
                    return c2

                lax.fori_loop(0, KG, fire, 0)
                pltpu.make_async_copy(
                    out_hbm.at[pl.ds(0, KG * CHUNK)],
                    rows_v.at[pl.ds(g * KG * CHUNK, KG * CHUNK)], sem).wait()
                return carry

            lax.fori_loop(0, NCH // KG, gbody, 0)
            pltpu.sync_copy(rows_v, out_hbm.at[pl.ds(wid * EPW, EPW)])

        gather(src_pinned_hbm, net_hbm, gpinned_hbm)
        gather(src_near_hbm, node_hbm, gnear_hbm)

    return k(net_feat, node_feat, src_pinned, src_near)


def _sc_agg_pins(x32, src_pins, dst_pins, zeros32):
    mesh = plsc.VectorSubcoreMesh(core_axis_name="c", subcore_axis_name="s")

    @functools.partial(
        pl.kernel,
        out_type=[jax.ShapeDtypeStruct((NC, NP, W), jnp.float32)],
        mesh=mesh,
        scratch_types=[
            pltpu.VMEM((NCH, CHUNK), jnp.int32),
            pltpu.VMEM((NCH, CHUNK), jnp.int32),
            pltpu.VMEM((KG * CHUNK, W), jnp.float32),
            pltpu.VMEM_SHARED((NP, W), jnp.float32),
            pltpu.SemaphoreType.DMA,
        ],
        compiler_params=pltpu.CompilerParams(use_tc_tiling_on_sc=False),
    )
    def k(x32_hbm, src_pins_hbm, dst_pins_hbm, zeros_hbm, out_hbm,
          idx_s, idx_d, rows_v, acc, sem):
        cid = lax.axis_index("c")
        sid = lax.axis_index("s")
        wid = sid * NC + cid

        pltpu.sync_copy(zeros_hbm.at[pl.ds(sid * RPT, RPT)],
                        acc.at[pl.ds(sid * RPT, RPT)])
        plsc.subcore_barrier()
        pltpu.sync_copy(src_pins_hbm.at[wid], idx_s)
        pltpu.sync_copy(dst_pins_hbm.at[wid], idx_d)

        def drain(n_rows):
            pltpu.make_async_copy(x32_hbm.at[pl.ds(0, n_rows)],
                                  rows_v.at[pl.ds(0, n_rows)], sem).wait()

        def pphase(q, carry):
            def fire_g(j, c2):
                pltpu.async_copy(x32_hbm.at[idx_s.at[q * KG + j]],
                                 rows_v.at[pl.ds(j * CHUNK, CHUNK)], sem)
                return c2

            lax.fori_loop(0, KG, fire_g, 0)
            drain(KG * CHUNK)

            def fire_s(j, c2):
                pltpu.async_copy(rows_v.at[pl.ds(j * CHUNK, CHUNK)],
                                 acc.at[idx_d.at[q * KG + j]], sem, add=True)
                return c2

            lax.fori_loop(0, KG, fire_s, 0)
            drain(KG * CHUNK)
            return carry

        lax.fori_loop(0, NCH // KG, pphase, 0)
        plsc.subcore_barrier()
        pltpu.sync_copy(acc.at[pl.ds(sid * RPT, RPT)],
                        out_hbm.at[cid, pl.ds(sid * RPT, RPT)])

    (out,) = k(x32, src_pins, dst_pins, zeros32)
    return out


def _sc_agg_msg(msg_p, msg_n, dst_pinned, dst_near, zeros32):
    mesh = plsc.VectorSubcoreMesh(core_axis_name="c", subcore_axis_name="s")

    @functools.partial(
        pl.kernel,
        out_type=[jax.ShapeDtypeStruct((NC, 2, NP, W), jnp.float32)],
        mesh=mesh,
        scratch_types=[
            pltpu.VMEM((NCH, CHUNK), jnp.int32),
            pltpu.VMEM((HROWS, W), jnp.float32),
            pltpu.VMEM_SHARED((NP, W), jnp.float32),
            pltpu.SemaphoreType.DMA,
        ],
        compiler_params=pltpu.CompilerParams(use_tc_tiling_on_sc=False),
    )
    def k(msg_p_hbm, msg_n_hbm, dst_pinned_hbm, dst_near_hbm, zeros_hbm,
          out_hbm, idx_d, rows_v, acc, sem):
        cid = lax.axis_index("c")
        sid = lax.axis_index("s")
        wid = sid * NC + cid

        def zero_acc():
            pltpu.sync_copy(zeros_hbm.at[pl.ds(sid * RPT, RPT)],
                            acc.at[pl.ds(sid * RPT, RPT)])
            plsc.subcore_barrier()

        def flush_acc(r):
            plsc.subcore_barrier()
            pltpu.sync_copy(acc.at[pl.ds(sid * RPT, RPT)],
                            out_hbm.at[cid, r, pl.ds(sid * RPT, RPT)])

        def scat(msg_hbm):
            def hbody(h, carry):
                pltpu.sync_copy(
                    msg_hbm.at[pl.ds(wid * EPW + h * HROWS, HROWS)], rows_v)

                def fire_s(j, c2):
                    pltpu.async_copy(rows_v.at[pl.ds(j * CHUNK, CHUNK)],
                                     acc.at[idx_d.at[h * HCH + j]], sem,
                                     add=True)
                    return c2

                lax.fori_loop(0, HCH, fire_s, 0)
                pltpu.make_async_copy(msg_hbm.at[pl.ds(0, HROWS)],
                                      rows_v, sem).wait()
                return carry

            lax.fori_loop(0, EPW // HROWS, hbody, 0)

        zero_acc()
        pltpu.sync_copy(dst_pinned_hbm.at[wid], idx_d)
        scat(msg_p_hbm)
        flush_acc(0)

        zero_acc()
        pltpu.sync_copy(dst_near_hbm.at[wid], idx_d)
        scat(msg_n_hbm)
        flush_acc(1)

    (out,) = k(msg_p, msg_n, dst_pinned, dst_near, zeros32)
    return out


def _tc_scale(nf_pad, deg_parts):
    def body(nf_ref, d_ref, o_ref):
        deg8 = d_ref[0] + d_ref[1]
        deg = jnp.concatenate([deg8, deg8], axis=1)
        x16 = nf_ref[...] * lax.rsqrt(jnp.maximum(deg, 1.0))
        o_ref[...] = jnp.concatenate(
            [x16, jnp.ones((NP, 8), jnp.float32)], axis=1)

    return pl.pallas_call(
        body, out_shape=jax.ShapeDtypeStruct((NP, W), jnp.float32),
    )(nf_pad, deg_parts)


def _tc_msg(g_p, g_n, ef_p, ef_n, w_p, b_p, w_n, b_n, r_c, t_c, blk):

    nblk = E // blk

    def body(gp_ref, gn_ref, efp_ref, efn_ref, wp_ref, bp_ref, wn_ref,
             bn_ref, r_ref, t_ref, op_ref, on_ref):
        ones = (lax.broadcasted_iota(jnp.int32, (blk, W), 1) >= 16).astype(
            jnp.float32)

        def msg(g_r, ef_r, w_r, b_r, o_r):
            w_e = jnp.dot(ef_r[...], w_r[...],
                          preferred_element_type=jnp.float32) + b_r[...]
            fx = jnp.dot(g_r[...], r_ref[...],
                         preferred_element_type=jnp.float32)
            m = jnp.dot(w_e * fx, t_ref[...],
                        preferred_element_type=jnp.float32)
            o_r[...] = m + ones

        msg(gp_ref, efp_ref, wp_ref, bp_ref, op_ref)
        msg(gn_ref, efn_ref, wn_ref, bn_ref, on_ref)

    edge_spec = pl.BlockSpec((blk, 16), lambda i: (i, 0))
    const_specs = [
        pl.BlockSpec((16, 256), lambda i: (0, 0)),
        pl.BlockSpec((1, 256), lambda i: (0, 0)),
        pl.BlockSpec((16, 256), lambda i: (0, 0)),
        pl.BlockSpec((1, 256), lambda i: (0, 0)),
        pl.BlockSpec((16, 256), lambda i: (0, 0)),
        pl.BlockSpec((256, W), lambda i: (0, 0)),
    ]
    out_spec = pl.BlockSpec((blk, W), lambda i: (i, 0))
    return pl.pallas_call(
        body,
        grid=(nblk,),
        in_specs=[edge_spec] * 4 + const_specs,
        out_specs=[out_spec, out_spec],
        out_shape=[jax.ShapeDtypeStruct((EP, W), jnp.float32),
                   jax.ShapeDtypeStruct((EP, W), jnp.float32)],
    )(g_p, g_n, ef_p, ef_n, w_p, b_p, w_n, b_n, r_c, t_c)


def _tc_final(parts_pins, parts_msg, w_gc, b_gc, b_pinned, b_near):
    def body(pp_ref, pm_ref, w_ref, bg_ref, bp_ref, bn_ref, node_ref,
             net_ref):
        def cnt16(c8):
            return jnp.concatenate([c8, c8], axis=1)

        agg = pp_ref[0, :, :16] + pp_ref[1, :, :16]
        c0 = cnt16(pp_ref[0, :, 16:W] + pp_ref[1, :, 16:W])
        rst = agg * lax.rsqrt(jnp.maximum(c0, 1.0))
        net_ref[...] = jnp.dot(rst, w_ref[...],
                               preferred_element_type=jnp.float32) + bg_ref[...]
        s1 = pm_ref[0, 0, :, :16] + pm_ref[1, 0, :, :16]
        c1 = cnt16(pm_ref[0, 0, :, 16:W] + pm_ref[1, 0, :, 16:W])
        o1 = s1 / jnp.maximum(c1, 1.0) + bp_ref[...]
        s2 = pm_ref[0, 1, :, :16] + pm_ref[1, 1, :, :16]
        c2 = cnt16(pm_ref[0, 1, :, 16:W] + pm_ref[1, 1, :, 16:W])
        o2 = s2 / jnp.maximum(c2, 1.0) + bn_ref[...]
        node_ref[...] = jnp.maximum(o1, o2)

    blk = NP // 8
    out_spec = pl.BlockSpec((blk, 16), lambda i: (i, 0))
    return pl.pallas_call(
        body,
        grid=(8,),
        in_specs=[
            pl.BlockSpec((NC, blk, W), lambda i: (0, i, 0)),
            pl.BlockSpec((NC, 2, blk, W), lambda i: (0, 0, i, 0)),
            pl.BlockSpec((16, 16), lambda i: (0, 0)),
            pl.BlockSpec((1, 16), lambda i: (0, 0)),
            pl.BlockSpec((1, 16), lambda i: (0, 0)),
            pl.BlockSpec((1, 16), lambda i: (0, 0)),
        ],
        out_specs=[out_spec, out_spec],
        out_shape=[jax.ShapeDtypeStruct((NP, 16), jnp.float32),
                   jax.ShapeDtypeStruct((NP, 16), jnp.float32)],
    )(parts_pins, parts_msg, w_gc, b_gc, b_pinned, b_near)


def kernel(node_feat, net_feat, pin_feat, edge_feat, pins_edge_index,
           pinned_edge_index, near_edge_index, w_gc, b_gc, w_topo, b_topo,
           w_geom, b_geom, b_pinned, b_near):
    f32 = jnp.float32

    def prep_idx(a, fill):
        pad = jnp.full((EP - E,), fill, jnp.int32)
        return jnp.concatenate([a.astype(jnp.int32), pad]).reshape(
            NW, NCH, CHUNK)

    def pad_rows(a, n):
        return jnp.concatenate(
            [a, jnp.zeros((n - a.shape[0], a.shape[1]), a.dtype)])

    src_pins = prep_idx(pins_edge_index[0], SENT)
    dst_pins = prep_idx(pins_edge_index[1], SENT)
    src_pinned = prep_idx(pinned_edge_index[0], 0)
    dst_pinned = prep_idx(pinned_edge_index[1], SENT)
    src_near = prep_idx(near_edge_index[0], 0)
    dst_near = prep_idx(near_edge_index[1], SENT)

    zeros24 = jnp.zeros((NP, W), f32)
    zeros8 = jnp.zeros((NP, 8), f32)
    ones8 = jnp.ones((CHUNK, 8), f32)
    nf_pad = pad_rows(node_feat, NP)
    r_c = jnp.asarray(_R_NP)
    t_c = jnp.asarray(_T24_NP)

    deg_parts = _sc_deg(src_pins, zeros8, ones8)
    gpinned, gnear = _sc_gather(net_feat, node_feat, src_pinned, src_near)
    x32 = _tc_scale(nf_pad, deg_parts)
    msg_p, msg_n = _tc_msg(gpinned, gnear, pin_feat, edge_feat, w_topo,
                           b_topo.reshape(1, 256), w_geom,
                           b_geom.reshape(1, 256), r_c, t_c, 8000)
    parts_pins = _sc_agg_pins(x32, src_pins, dst_pins, zeros24)
    parts_msg = _sc_agg_msg(msg_p, msg_n, dst_pinned, dst_near, zeros24)
    node_out, net_out = _tc_final(parts_pins, parts_msg, w_gc,
                                  b_gc.reshape(1, 16),
                                  b_pinned.reshape(1, 16),
                                  b_near.reshape(1, 16))
    return node_out[:N], net_out[:N]
